# Initial kernel scaffold; baseline (speedup 1.0000x reference)
import jax
import jax.numpy as jnp
from jax.experimental import pallas as pl


def kernel(x, edge_index, treat_idx, control_idx, W1, b1, W2, b2, Wy1, by1, Wy0, by0):
    raise NotImplementedError("write your pallas kernel here")


# speedup vs baseline: 15.6550x; 15.6550x over previous
import functools

import jax
import jax.numpy as jnp
from jax import lax
from jax.experimental import pallas as pl
from jax.experimental.pallas import tpu as pltpu
from jax.experimental.pallas import tpu_sc as plsc

_N = 10000
_E = 320000
_D = 128
_NPAD = 10240
_NTILES = 32
_EPT = _E // _NTILES
_CHUNK = 128
_FULL = _EPT // _CHUNK
_REM = _EPT - _FULL * _CHUNK


def _sc_mesh():
    return plsc.VectorSubcoreMesh(core_axis_name="c", subcore_axis_name="s")


@functools.partial(
    pl.kernel,
    out_type=jax.ShapeDtypeStruct((2 * _NPAD,), jnp.float32),
    mesh=_sc_mesh(),
    scratch_types=[
        pltpu.VMEM_SHARED((_NPAD,), jnp.float32),
        pltpu.VMEM((_CHUNK,), jnp.int32),
        pltpu.VMEM((_REM,), jnp.int32),
        pltpu.VMEM((_CHUNK,), jnp.float32),
        pltpu.VMEM((_REM,), jnp.float32),
    ],
)
def _deg_kernel(dst_hbm, zeros_hbm, out_hbm, acc, idx, idx16, ones, ones16):
    c = lax.axis_index("c")
    s = lax.axis_index("s")
    r0 = s * (_NPAD // 16)
    pltpu.sync_copy(zeros_hbm.at[pl.ds(r0, _NPAD // 16)],
                    acc.at[pl.ds(r0, _NPAD // 16)])
    for j in range(_CHUNK // 16):
        ones[pl.ds(j * 16, 16)] = jnp.ones((16,), jnp.float32)
    ones16[...] = jnp.ones((_REM,), jnp.float32)
    plsc.subcore_barrier()

    base = (c * 16 + s) * _EPT

    def body(i, carry):
        pltpu.sync_copy(dst_hbm.at[pl.ds(base + i * _CHUNK, _CHUNK)], idx)
        pltpu.sync_copy(ones, acc.at[idx], add=True)
        return carry

    lax.fori_loop(0, _FULL, body, 0)
    pltpu.sync_copy(dst_hbm.at[pl.ds(base + _FULL * _CHUNK, _REM)], idx16)
    pltpu.sync_copy(ones16, acc.at[idx16], add=True)
    plsc.subcore_barrier()
    pltpu.sync_copy(acc.at[pl.ds(r0, _NPAD // 16)],
                    out_hbm.at[pl.ds(c * _NPAD + r0, _NPAD // 16)])


@functools.partial(
    pl.kernel,
    out_type=jax.ShapeDtypeStruct((2, _NPAD, _D), jnp.float32),
    mesh=_sc_mesh(),
    scratch_types=[
        pltpu.VMEM_SHARED((_NPAD, _D), jnp.float32),
        pltpu.VMEM((_CHUNK,), jnp.int32),
        pltpu.VMEM((_CHUNK,), jnp.int32),
        pltpu.VMEM((_REM,), jnp.int32),
        pltpu.VMEM((_REM,), jnp.int32),
        pltpu.VMEM((_CHUNK, _D), jnp.float32),
        pltpu.VMEM((_REM, _D), jnp.float32),
        pltpu.SemaphoreType.DMA,
    ],
)
def _scatter_kernel(src_hbm, dst_hbm, g_hbm, zeros_hbm, out_hbm,
                    acc, sidx, didx, sidx16, didx16, rows, rows16, sem):
    c = lax.axis_index("c")
    s = lax.axis_index("s")
    npt = _NPAD // 16
    r0 = s * npt
    @pl.when(jnp.logical_and(c == 0, s < 15))
    def _():
        pltpu.sync_copy(g_hbm.at[pl.ds(r0, npt)], acc.at[pl.ds(r0, npt)])

    @pl.when(jnp.logical_and(c == 0, s == 15))
    def _():
        pltpu.sync_copy(g_hbm.at[pl.ds(15 * npt, _N - 15 * npt)],
                        acc.at[pl.ds(15 * npt, _N - 15 * npt)])

    @pl.when(jnp.logical_and(c != 0, s < 15))
    def _():
        pltpu.sync_copy(zeros_hbm.at[pl.ds(r0, npt)], acc.at[pl.ds(r0, npt)])

    @pl.when(jnp.logical_and(c != 0, s == 15))
    def _():
        pltpu.sync_copy(zeros_hbm.at[pl.ds(15 * npt, _N - 15 * npt)],
                        acc.at[pl.ds(15 * npt, _N - 15 * npt)])

    plsc.subcore_barrier()
    base = (c * 16 + s) * _EPT

    def body(i, carry):
        off = base + i * _CHUNK
        pltpu.sync_copy(src_hbm.at[pl.ds(off, _CHUNK)], sidx)
        pltpu.sync_copy(dst_hbm.at[pl.ds(off, _CHUNK)], didx)
        pltpu.async_copy(g_hbm.at[sidx], rows, sem).wait()
        pltpu.sync_copy(rows, acc.at[didx], add=True)
        return carry

    lax.fori_loop(0, _FULL, body, 0)
    off = base + _FULL * _CHUNK
    pltpu.sync_copy(src_hbm.at[pl.ds(off, _REM)], sidx16)
    pltpu.sync_copy(dst_hbm.at[pl.ds(off, _REM)], didx16)
    pltpu.async_copy(g_hbm.at[sidx16], rows16, sem).wait()
    pltpu.sync_copy(rows16, acc.at[didx16], add=True)
    plsc.subcore_barrier()
    pltpu.sync_copy(acc.at[pl.ds(r0, npt)], out_hbm.at[c, pl.ds(r0, npt)])


_NIDX = 10240
_IPT = _NIDX // _NTILES


@functools.partial(
    pl.kernel,
    out_type=jax.ShapeDtypeStruct((_NIDX, _D), jnp.float32),
    mesh=_sc_mesh(),
    scratch_types=[
        pltpu.VMEM((_CHUNK,), jnp.int32),
        pltpu.VMEM((64,), jnp.int32),
        pltpu.VMEM((_CHUNK, _D), jnp.float32),
        pltpu.VMEM((64, _D), jnp.float32),
        pltpu.SemaphoreType.DMA,
    ],
)
def _headgather_kernel(v_hbm, idx_hbm, out_hbm, idx128, idx64, rows, rows64, sem):
    c = lax.axis_index("c")
    s = lax.axis_index("s")
    base = (c * 16 + s) * _IPT
    for j in range(2):
        pltpu.sync_copy(idx_hbm.at[pl.ds(base + j * _CHUNK, _CHUNK)], idx128)
        pltpu.async_copy(v_hbm.at[idx128], rows, sem).wait()
        pltpu.sync_copy(rows, out_hbm.at[pl.ds(base + j * _CHUNK, _CHUNK)])
    pltpu.sync_copy(idx_hbm.at[pl.ds(base + 256, 64)], idx64)
    pltpu.async_copy(v_hbm.at[idx64], rows64, sem).wait()
    pltpu.sync_copy(rows64, out_hbm.at[pl.ds(base + 256, 64)])


_BN = 1000


def _tc1_body(x_ref, w_ref, deg_ref, g_ref, dinv_ref):
    d = deg_ref[:, 0:1] + deg_ref[:, 1:2]
    dinv = lax.rsqrt(d + 1.0)
    h = jnp.dot(x_ref[...], w_ref[...], preferred_element_type=jnp.float32)
    g_ref[...] = h * dinv
    dinv_ref[...] = dinv


def _tc1(x, W1, degT):
    return pl.pallas_call(
        _tc1_body,
        grid=(_N // _BN,),
        in_specs=[
            pl.BlockSpec((_BN, _D), lambda i: (i, 0)),
            pl.BlockSpec((_D, _D), lambda i: (0, 0)),
            pl.BlockSpec((_BN, 2), lambda i: (i, 0)),
        ],
        out_specs=[
            pl.BlockSpec((_BN, _D), lambda i: (i, 0)),
            pl.BlockSpec((_BN, 1), lambda i: (i, 0)),
        ],
        out_shape=[
            jax.ShapeDtypeStruct((_N, _D), jnp.float32),
            jax.ShapeDtypeStruct((_N, 1), jnp.float32),
        ],
    )(x, W1, degT)


def _tc2_body(accA_ref, accB_ref, dinv_ref, b_ref, w_ref, g_ref):
    dinv = dinv_ref[...]
    pre = (accA_ref[0] + accB_ref[0]) * dinv + b_ref[...]
    xz1 = jnp.maximum(pre, 0.0)
    g_ref[...] = jnp.dot(xz1, w_ref[...],
                         preferred_element_type=jnp.float32) * dinv


def _tc2(acc, dinv, b1, W2):
    return pl.pallas_call(
        _tc2_body,
        grid=(_N // _BN,),
        in_specs=[
            pl.BlockSpec((1, _BN, _D), lambda i: (0, i, 0)),
            pl.BlockSpec((1, _BN, _D), lambda i: (1, i, 0)),
            pl.BlockSpec((_BN, 1), lambda i: (i, 0)),
            pl.BlockSpec((1, _D), lambda i: (0, 0)),
            pl.BlockSpec((_D, _D), lambda i: (0, 0)),
        ],
        out_specs=pl.BlockSpec((_BN, _D), lambda i: (i, 0)),
        out_shape=jax.ShapeDtypeStruct((_N, _D), jnp.float32),
    )(acc, acc, dinv, b1, W2)


def _tc3_body(accA_ref, accB_ref, dinv_ref, b_ref, wy_ref, by_ref,
              xz2_ref, v_ref):
    xz2 = (accA_ref[0] + accB_ref[0]) * dinv_ref[...] + b_ref[...]
    xz2_ref[...] = xz2
    u = jnp.dot(xz2, wy_ref[...], preferred_element_type=jnp.float32) + by_ref[...]
    v_ref[...] = jnp.where(u >= 0, u, 0.01 * u)


def _tc3(acc, dinv, b2, wy, by):
    return pl.pallas_call(
        _tc3_body,
        grid=(_N // _BN,),
        in_specs=[
            pl.BlockSpec((1, _BN, _D), lambda i: (0, i, 0)),
            pl.BlockSpec((1, _BN, _D), lambda i: (1, i, 0)),
            pl.BlockSpec((_BN, 1), lambda i: (i, 0)),
            pl.BlockSpec((1, _D), lambda i: (0, 0)),
            pl.BlockSpec((_D, _D), lambda i: (0, 0)),
            pl.BlockSpec((1, _D), lambda i: (0, 0)),
        ],
        out_specs=[
            pl.BlockSpec((_BN, _D), lambda i: (i, 0)),
            pl.BlockSpec((_BN, _D), lambda i: (i, 0)),
        ],
        out_shape=[
            jax.ShapeDtypeStruct((_N, _D), jnp.float32),
            jax.ShapeDtypeStruct((_N, _D), jnp.float32),
        ],
    )(acc, acc, dinv, b2, wy, by)


def kernel(x, edge_index, treat_idx, control_idx,
           W1, b1, W2, b2, Wy1, by1, Wy0, by0):
    f32 = jnp.float32
    src = edge_index[0]
    dst = edge_index[1]
    zeros_big = jnp.zeros((_N, _D), f32)
    zeros_deg = jnp.zeros((_NPAD,), f32)

    deg_parts = _deg_kernel(dst, zeros_deg)
    degT = deg_parts.reshape(2, _NPAD).T

    g1, dinv = _tc1(x, W1, degT)
    acc1 = _scatter_kernel(src, dst, g1, zeros_big)
    g2 = _tc2(acc1, dinv, b1.reshape(1, _D), W2)
    acc2 = _scatter_kernel(src, dst, g2, zeros_big)

    wy = jnp.zeros((_D, _D), f32).at[:, 0:1].set(Wy1).at[:, 1:2].set(Wy0)
    by = jnp.zeros((_D,), f32).at[0].set(by1[0]).at[1].set(by0[0]).reshape(1, _D)
    xZ2, v = _tc3(acc2, dinv, b2.reshape(1, _D), wy, by)

    idx_all = jnp.concatenate(
        [treat_idx, control_idx,
         jnp.zeros((_NIDX - 2 * 5000,), jnp.int32)])
    yout = _headgather_kernel(v, idx_all)

    y1 = yout[:5000, 0]
    yc0 = yout[:5000, 1]
    y0 = yout[5000:10000, 1]
    yc1 = yout[5000:10000, 0]
    return (y1, yc0, y0, yc1, xZ2)
